# Initial kernel scaffold; baseline (speedup 1.0000x reference)
import jax
import jax.numpy as jnp
from jax.experimental import pallas as pl


def kernel(points, params):
    raise NotImplementedError("write your pallas kernel here")


# speedup vs baseline: 2.3181x; 2.3181x over previous
import functools

import jax
import jax.numpy as jnp
from jax import lax
from jax.experimental import pallas as pl
from jax.experimental.pallas import tpu as pltpu

F32 = jnp.float32
_K = 20
_E = 64


def _dot_nt(a, b, precision=lax.Precision.HIGHEST):
    return lax.dot_general(a, b, (((1,), (1,)), ((), ())),
                           preferred_element_type=F32, precision=precision)


def _dot_nn(a, b, precision=lax.Precision.HIGHEST):
    return lax.dot_general(a, b, (((1,), (0,)), ((), ())),
                           preferred_element_type=F32, precision=precision)


def _relu(x):
    return jnp.maximum(x, 0.0)


def _erf(x):
    a1, a2, a3, a4, a5 = (0.254829592, -0.284496736, 1.421413741,
                          -1.453152027, 1.061405429)
    t = 1.0 / (1.0 + 0.3275911 * jnp.abs(x))
    poly = ((((a5 * t + a4) * t + a3) * t + a2) * t + a1) * t
    y = 1.0 - poly * jnp.exp(-x * x)
    return jnp.sign(x) * y


def _gelu(x):
    return 0.5 * x * (1.0 + _erf(x * 0.7071067811865476))


def _layer_norm(x, g, b):
    m = jnp.mean(x, axis=1, keepdims=True)
    v = jnp.mean((x - m) ** 2, axis=1, keepdims=True)
    return (x - m) / jnp.sqrt(v + 1e-5) * g + b


def _fps(px, py, pz, n_out, o_ref):
    n_in = px.shape[1]
    iota = lax.broadcasted_iota(jnp.int32, (1, n_in), 1).astype(F32)

    def step(s, carry):
        dists, far = carry
        oh = (iota == far).astype(F32)
        o_ref[pl.ds(s, 1), :] = oh
        sx = jnp.sum(oh * px)
        sy = jnp.sum(oh * py)
        sz = jnp.sum(oh * pz)
        d = (px - sx) ** 2 + (py - sy) ** 2 + (pz - sz) ** 2
        dists = jnp.minimum(dists, d)
        far = jnp.min(jnp.where(dists == jnp.max(dists), iota, float(n_in)))
        return dists, far

    lax.fori_loop(0, n_out, step,
                  (jnp.full((1, n_in), 1e10, F32), jnp.float32(0.0)))


def _pair_dist(cr_list, cc_list):
    n = cr_list[0].shape[1]
    d = ((cc_list[0] - cr_list[0]) ** 2
         + (cc_list[1] - cr_list[1]) ** 2
         + (cc_list[2] - cr_list[2]) ** 2)
    rio = lax.broadcasted_iota(jnp.int32, (n, n), 0)
    cio = lax.broadcasted_iota(jnp.int32, (n, n), 1)
    return d + (rio == cio).astype(F32) * 1e10


def _knn_edge(dmat, x, wmat, bias):
    n = dmat.shape[0]
    cout = wmat.shape[0]
    cio = lax.broadcasted_iota(jnp.int32, (n, n), 1).astype(F32)
    hi = lax.Precision.HIGHEST

    def step(_, carry):
        d, m = carry
        rmin = jnp.min(d, axis=1, keepdims=True)
        nbr = jnp.min(jnp.where(d == rmin, cio, float(n)),
                      axis=1, keepdims=True)
        oh = (cio == nbr).astype(F32)
        xj = _dot_nn(oh, x, hi)
        cat = jnp.concatenate([x, xj - x], axis=1)
        msg = _relu(_dot_nt(cat, wmat, lax.Precision.DEFAULT) + bias)
        return d + oh * 1e30, jnp.maximum(m, msg)

    _, m = lax.fori_loop(0, _K, step,
                         (dmat, jnp.full((n, cout), -1e30, F32)))
    return m


def _stage(coords, feats, n_out, wmat, bias, o_ref):
    px, py, pz = coords
    _fps(px, py, pz, n_out, o_ref)
    onehot = o_ref[...]
    hi = lax.Precision.HIGHEST
    xg = _dot_nn(onehot, feats, hi)
    crs = [_dot_nt(c, onehot, hi) for c in coords]
    ccs = [_dot_nt(onehot, c, hi) for c in coords]
    dmat = _pair_dist(crs, ccs)
    xe = _knn_edge(dmat, xg, wmat, bias)
    return crs, jnp.concatenate([xg, xe], axis=1)


def _cross_block(s, w, i, prec):
    u = _dot_nt(s, w[f'sa{i}_ip_w'], prec) + w[f'sa{i}_ip_b']
    ln1 = _layer_norm(u, w[f'sa{i}_n13_g'], w[f'sa{i}_n13_b'])
    vproj = _dot_nt(ln1, w[f'sa{i}_vw'], prec) + w[f'sa{i}_vb']
    vproj = vproj.astype(jnp.bfloat16).astype(F32)
    attn = _dot_nt(vproj, w[f'sa{i}_out_w'], prec) + w[f'sa{i}_out_b']
    s1 = ln1 + attn
    ln2 = _layer_norm(s1, w[f'sa{i}_n12_g'], w[f'sa{i}_n12_b'])
    h = _gelu(_dot_nt(ln2, w[f'sa{i}_l11_w'], prec) + w[f'sa{i}_l11_b'])
    ff = _dot_nt(h, w[f'sa{i}_l12_w'], prec) + w[f'sa{i}_l12_b']
    return ln2 + ff


def _body(pts_ref, ptsr_ref, w_refs, xg_ref, f3_ref, o0_ref, o1_ref, o2_ref):
    w = {k: r[...] for k, r in w_refs.items()}
    pts = pts_ref[0]
    px = pts[0:1, :]
    py = pts[1:2, :]
    pz = pts[2:3, :]
    ptsr = ptsr_ref[0]

    lo = lax.Precision.DEFAULT
    h = _relu(_dot_nt(ptsr, w['W1'], lo) + w['b1'])
    x0 = _dot_nt(h, w['W2'], lo) + w['b2']

    n = px.shape[1]
    c0, f1 = _stage((px, py, pz), x0, n // 4,
                    w['ec1_W'], w['ec1_b'], o0_ref)
    c1, f2 = _stage(c0, f1, n // 8,
                    w['ec2_W'], w['ec2_b'], o1_ref)
    _, f3 = _stage(c1, f2, n // 16,
                   w['ec3_W'], w['ec3_b'], o2_ref)

    g = jnp.max(f3, axis=0, keepdims=True)
    xg_ref[0] = g

    v = _relu(_dot_nt(g, w['Wpa'], lo) + w['bpa'])
    v = _relu(_dot_nt(v, w['Wps'], lo) + w['bps'])
    v = _relu(_dot_nt(v, w['Wpr'], lo) + w['bpr'])
    for i in range(3):
        v = _cross_block(v, w, i, lo)
    c = _relu(_dot_nt(v, w['Wco1'], lo) + w['bco1'])
    f3_ref[0] = _dot_nt(c, w['Wco'], lo) + w['bco']


@jax.jit
def kernel(points, params):
    p = params
    B, _, N = points.shape
    E = _E

    wd = {
        'W1': p['W1'], 'b1': p['b1'].reshape(1, -1),
        'W2': p['W2'], 'b2': p['b2'].reshape(1, -1),
        'Wpa': p['Wpa'], 'bpa': p['bpa'].reshape(1, -1),
        'Wps': p['Wps'], 'bps': p['bps'].reshape(1, -1),
        'Wpr': p['Wpr'], 'bpr': p['bpr'].reshape(1, -1),
        'Wco1': p['Wco1'], 'bco1': p['bco1'].reshape(1, -1),
        'Wco': p['Wco'], 'bco': p['bco'].reshape(1, -1),
    }
    for name in ('ec1', 'ec2', 'ec3'):
        wd[f'{name}_W'] = p[f'{name}_W']
        wd[f'{name}_b'] = p[f'{name}_b'].reshape(1, -1)
    for i in range(3):
        sa = p[f'sa{i}']
        wd[f'sa{i}_ip_w'] = sa['ip_w']
        wd[f'sa{i}_ip_b'] = sa['ip_b'].reshape(1, -1)
        wd[f'sa{i}_vw'] = sa['in_w'][2 * E:3 * E]
        wd[f'sa{i}_vb'] = sa['in_b'][2 * E:3 * E].reshape(1, -1)
        wd[f'sa{i}_out_w'] = sa['out_w']
        wd[f'sa{i}_out_b'] = sa['out_b'].reshape(1, -1)
        wd[f'sa{i}_l11_w'] = sa['l11_w']
        wd[f'sa{i}_l11_b'] = sa['l11_b'].reshape(1, -1)
        wd[f'sa{i}_l12_w'] = sa['l12_w']
        wd[f'sa{i}_l12_b'] = sa['l12_b'].reshape(1, -1)
        wd[f'sa{i}_n12_g'] = sa['n12_g'].reshape(1, -1)
        wd[f'sa{i}_n12_b'] = sa['n12_b'].reshape(1, -1)
        wd[f'sa{i}_n13_g'] = sa['n13_g'].reshape(1, -1)
        wd[f'sa{i}_n13_b'] = sa['n13_b'].reshape(1, -1)

    ptsr = jnp.transpose(points, (0, 2, 1))

    def full_spec(a):
        return pl.BlockSpec(a.shape, lambda b, nd=a.ndim: (0,) * nd)

    in_specs = [
        pl.BlockSpec((1, 3, N), lambda b: (b, 0, 0)),
        pl.BlockSpec((1, N, 3), lambda b: (b, 0, 0)),
        {k: full_spec(v) for k, v in wd.items()},
    ]
    out_specs = [
        pl.BlockSpec((1, 1, 1024), lambda b: (b, 0, 0)),
        pl.BlockSpec((1, 1, 3), lambda b: (b, 0, 0)),
    ]
    out_shape = [
        jax.ShapeDtypeStruct((B, 1, 1024), F32),
        jax.ShapeDtypeStruct((B, 1, 3), F32),
    ]
    scratch_shapes = [
        pltpu.VMEM((N // 4, N), F32),
        pltpu.VMEM((N // 8, N // 4), F32),
        pltpu.VMEM((N // 16, N // 8), F32),
    ]

    xg, f3 = pl.pallas_call(
        _body,
        grid=(B,),
        in_specs=in_specs,
        out_specs=out_specs,
        out_shape=out_shape,
        scratch_shapes=scratch_shapes,
    )(points, ptsr, wd)

    fine = jnp.broadcast_to(f3[:, 0, :, None], (B, 3, 128))
    return xg[:, 0, :, None], fine


# speedup vs baseline: 2.5943x; 1.1191x over previous
import jax
import jax.numpy as jnp
from jax import lax
from jax.experimental import pallas as pl
from jax.experimental.pallas import tpu as pltpu

F32 = jnp.float32
BF16 = jnp.bfloat16
_K = 20
_E = 64


def _dot_nt(a, b, precision=None):
    return lax.dot_general(a, b, (((1,), (1,)), ((), ())),
                           preferred_element_type=F32, precision=precision)


def _dot_nn(a, b, precision=None):
    return lax.dot_general(a, b, (((1,), (0,)), ((), ())),
                           preferred_element_type=F32, precision=precision)


def _relu(x):
    return jnp.maximum(x, 0.0)


def _split3(v):
    h = v.astype(BF16)
    r = v - h.astype(F32)
    m = r.astype(BF16)
    l = (r - m.astype(F32)).astype(BF16)
    return h, m, l


def _gather_exact(oh_b, parts):
    h, m, l = parts
    return (_dot_nn(oh_b, h) + _dot_nn(oh_b, m)) + _dot_nn(oh_b, l)


def _erf(x):
    a1, a2, a3, a4, a5 = (0.254829592, -0.284496736, 1.421413741,
                          -1.453152027, 1.061405429)
    t = 1.0 / (1.0 + 0.3275911 * jnp.abs(x))
    poly = ((((a5 * t + a4) * t + a3) * t + a2) * t + a1) * t
    y = 1.0 - poly * jnp.exp(-x * x)
    return jnp.sign(x) * y


def _gelu(x):
    return 0.5 * x * (1.0 + _erf(x * 0.7071067811865476))


def _layer_norm(x, g, b):
    m = jnp.mean(x, axis=1, keepdims=True)
    v = jnp.mean((x - m) ** 2, axis=1, keepdims=True)
    return (x - m) / jnp.sqrt(v + 1e-5) * g + b


def _fps(coords, iota, sentinel, n_out, idx_ref):
    cx, cy, cz = coords

    def step(s, carry):
        dists, far = carry
        oh = iota == far
        idx_ref[pl.ds(s, 1), :] = jnp.reshape(far, (1, 1))
        sx = jnp.sum(jnp.where(oh, cx, 0.0))
        sy = jnp.sum(jnp.where(oh, cy, 0.0))
        sz = jnp.sum(jnp.where(oh, cz, 0.0))
        d = (cx - sx) ** 2 + (cy - sy) ** 2 + (cz - sz) ** 2
        dists = jnp.minimum(dists, d)
        far = jnp.min(jnp.where(dists == jnp.max(dists), iota, sentinel))
        return dists, far

    lax.fori_loop(0, n_out, step,
                  (jnp.full(cx.shape, 1e10, F32), jnp.float32(0.0)))


def _pair_dist(cr_list, cc_list):
    n = cr_list[0].shape[1]
    d = ((cc_list[0] - cr_list[0]) ** 2
         + (cc_list[1] - cr_list[1]) ** 2
         + (cc_list[2] - cr_list[2]) ** 2)
    rio = lax.broadcasted_iota(jnp.int32, (n, n), 0)
    cio = lax.broadcasted_iota(jnp.int32, (n, n), 1)
    return d + (rio == cio).astype(F32) * 1e10


def _knn_edge(dmat, x, wa, wb, bias):
    n = dmat.shape[0]
    cout = wa.shape[0]
    cio = lax.broadcasted_iota(jnp.int32, (n, n), 1).astype(F32)
    lo = lax.Precision.DEFAULT
    q = _dot_nt(x, wa, lo) + bias
    parts = _split3(x)

    def step(_, carry):
        d, m = carry
        rmin = jnp.min(d, axis=1, keepdims=True)
        nbr = jnp.min(jnp.where(d == rmin, cio, float(n)),
                      axis=1, keepdims=True)
        oh = (cio == nbr).astype(F32)
        xj = _gather_exact(oh.astype(BF16), parts)
        msg = _relu(q + _dot_nt(xj - x, wb, lo))
        return d + oh * 1e30, jnp.maximum(m, msg)

    _, m = lax.fori_loop(0, _K, step,
                         (dmat, jnp.full((n, cout), -1e30, F32)))
    return m


def _stage(coords_fps, iota_fps, coords_row, feats, n_out,
           wa, wb, bias, idx_ref):
    n_in = coords_row[0].shape[1]
    _fps(coords_fps, iota_fps, float(n_in), n_out, idx_ref)
    sel = idx_ref[0:n_out, :]
    iota_row = lax.broadcasted_iota(jnp.int32, (1, n_in), 1).astype(F32)
    onehot = (sel == iota_row).astype(F32)
    xg = _gather_exact(onehot.astype(BF16), _split3(feats))
    hi = lax.Precision.HIGHEST
    crs = [_dot_nt(c, onehot, hi) for c in coords_row]
    ccs = [_dot_nt(onehot, c, hi) for c in coords_row]
    dmat = _pair_dist(crs, ccs)
    xe = _knn_edge(dmat, xg, wa, wb, bias)
    return crs, jnp.concatenate([xg, xe], axis=1)


def _cross_block(s, w, i, prec):
    u = _dot_nt(s, w[f'sa{i}_ip_w'], prec) + w[f'sa{i}_ip_b']
    ln1 = _layer_norm(u, w[f'sa{i}_n13_g'], w[f'sa{i}_n13_b'])
    vproj = _dot_nt(ln1, w[f'sa{i}_vw'], prec) + w[f'sa{i}_vb']
    vproj = vproj.astype(BF16).astype(F32)
    attn = _dot_nt(vproj, w[f'sa{i}_out_w'], prec) + w[f'sa{i}_out_b']
    s1 = ln1 + attn
    ln2 = _layer_norm(s1, w[f'sa{i}_n12_g'], w[f'sa{i}_n12_b'])
    h = _gelu(_dot_nt(ln2, w[f'sa{i}_l11_w'], prec) + w[f'sa{i}_l11_b'])
    ff = _dot_nt(h, w[f'sa{i}_l12_w'], prec) + w[f'sa{i}_l12_b']
    return ln2 + ff


def _body(pts_ref, pts8_ref, ptsr_ref, w_refs, xg_ref, f3_ref, idx_ref):
    w = {k: r[...] for k, r in w_refs.items()}
    pts = pts_ref[0]
    px = pts[0:1, :]
    py = pts[1:2, :]
    pz = pts[2:3, :]
    p8 = pts8_ref[0]
    ptsr = ptsr_ref[0]
    n = px.shape[1]
    nc = n // 8

    lo = lax.Precision.DEFAULT
    h = _relu(_dot_nt(ptsr, w['W1'], lo) + w['b1'])
    x0 = _dot_nt(h, w['W2'], lo) + w['b2']

    iota8 = (lax.broadcasted_iota(jnp.int32, (8, nc), 0) * nc
             + lax.broadcasted_iota(jnp.int32, (8, nc), 1)).astype(F32)

    c0, f1 = _stage((p8[0], p8[1], p8[2]), iota8, (px, py, pz), x0, n // 4,
                    w['ec1_Wa'], w['ec1_Wb'], w['ec1_b'], idx_ref)
    iota1 = lax.broadcasted_iota(jnp.int32, (1, n // 4), 1).astype(F32)
    c1, f2 = _stage(tuple(c0), iota1, tuple(c0), f1, n // 8,
                    w['ec2_Wa'], w['ec2_Wb'], w['ec2_b'], idx_ref)
    iota2 = lax.broadcasted_iota(jnp.int32, (1, n // 8), 1).astype(F32)
    _, f3 = _stage(tuple(c1), iota2, tuple(c1), f2, n // 16,
                   w['ec3_Wa'], w['ec3_Wb'], w['ec3_b'], idx_ref)

    g = jnp.max(f3, axis=0, keepdims=True)
    xg_ref[0] = g

    v = _relu(_dot_nt(g, w['Wpa'], lo) + w['bpa'])
    v = _relu(_dot_nt(v, w['Wps'], lo) + w['bps'])
    v = _relu(_dot_nt(v, w['Wpr'], lo) + w['bpr'])
    for i in range(3):
        v = _cross_block(v, w, i, lo)
    c = _relu(_dot_nt(v, w['Wco1'], lo) + w['bco1'])
    f3_ref[0] = _dot_nt(c, w['Wco'], lo) + w['bco']


@jax.jit
def kernel(points, params):
    p = params
    B, _, N = points.shape
    E = _E

    wd = {
        'W1': p['W1'], 'b1': p['b1'].reshape(1, -1),
        'W2': p['W2'], 'b2': p['b2'].reshape(1, -1),
        'Wpa': p['Wpa'], 'bpa': p['bpa'].reshape(1, -1),
        'Wps': p['Wps'], 'bps': p['bps'].reshape(1, -1),
        'Wpr': p['Wpr'], 'bpr': p['bpr'].reshape(1, -1),
        'Wco1': p['Wco1'], 'bco1': p['bco1'].reshape(1, -1),
        'Wco': p['Wco'], 'bco': p['bco'].reshape(1, -1),
    }
    for name in ('ec1', 'ec2', 'ec3'):
        W = p[f'{name}_W']
        c = W.shape[1] // 2
        wd[f'{name}_Wa'] = W[:, :c]
        wd[f'{name}_Wb'] = W[:, c:]
        wd[f'{name}_b'] = p[f'{name}_b'].reshape(1, -1)
    for i in range(3):
        sa = p[f'sa{i}']
        wd[f'sa{i}_ip_w'] = sa['ip_w']
        wd[f'sa{i}_ip_b'] = sa['ip_b'].reshape(1, -1)
        wd[f'sa{i}_vw'] = sa['in_w'][2 * E:3 * E]
        wd[f'sa{i}_vb'] = sa['in_b'][2 * E:3 * E].reshape(1, -1)
        wd[f'sa{i}_out_w'] = sa['out_w']
        wd[f'sa{i}_out_b'] = sa['out_b'].reshape(1, -1)
        wd[f'sa{i}_l11_w'] = sa['l11_w']
        wd[f'sa{i}_l11_b'] = sa['l11_b'].reshape(1, -1)
        wd[f'sa{i}_l12_w'] = sa['l12_w']
        wd[f'sa{i}_l12_b'] = sa['l12_b'].reshape(1, -1)
        wd[f'sa{i}_n12_g'] = sa['n12_g'].reshape(1, -1)
        wd[f'sa{i}_n12_b'] = sa['n12_b'].reshape(1, -1)
        wd[f'sa{i}_n13_g'] = sa['n13_g'].reshape(1, -1)
        wd[f'sa{i}_n13_b'] = sa['n13_b'].reshape(1, -1)

    pts8 = points.reshape(B, 3, 8, N // 8)
    ptsr = jnp.transpose(points, (0, 2, 1))

    def full_spec(a):
        return pl.BlockSpec(a.shape, lambda b, nd=a.ndim: (0,) * nd)

    in_specs = [
        pl.BlockSpec((1, 3, N), lambda b: (b, 0, 0)),
        pl.BlockSpec((1, 3, 8, N // 8), lambda b: (b, 0, 0, 0)),
        pl.BlockSpec((1, N, 3), lambda b: (b, 0, 0)),
        {k: full_spec(v) for k, v in wd.items()},
    ]
    out_specs = [
        pl.BlockSpec((1, 1, 1024), lambda b: (b, 0, 0)),
        pl.BlockSpec((1, 1, 3), lambda b: (b, 0, 0)),
    ]
    out_shape = [
        jax.ShapeDtypeStruct((B, 1, 1024), F32),
        jax.ShapeDtypeStruct((B, 1, 3), F32),
    ]
    scratch_shapes = [
        pltpu.VMEM((N // 4, 1), F32),
    ]

    xg, f3 = pl.pallas_call(
        _body,
        grid=(B,),
        in_specs=in_specs,
        out_specs=out_specs,
        out_shape=out_shape,
        scratch_shapes=scratch_shapes,
    )(points, pts8, ptsr, wd)

    fine = jnp.broadcast_to(f3[:, 0, :, None], (B, 3, 128))
    return xg[:, 0, :, None], fine


# speedup vs baseline: 2.6181x; 1.0092x over previous
import jax
import jax.numpy as jnp
from jax import lax
from jax.experimental import pallas as pl
from jax.experimental.pallas import tpu as pltpu

F32 = jnp.float32
BF16 = jnp.bfloat16
_K = 20
_E = 64
_LO = lax.Precision.DEFAULT


def _dot_nt(a, b, precision=None):
    return lax.dot_general(a, b, (((1,), (1,)), ((), ())),
                           preferred_element_type=F32, precision=precision)


def _dot_nn(a, b, precision=None):
    return lax.dot_general(a, b, (((1,), (0,)), ((), ())),
                           preferred_element_type=F32, precision=precision)


def _relu(x):
    return jnp.maximum(x, 0.0)


def _split3(v):
    h = v.astype(BF16)
    r = v - h.astype(F32)
    m = r.astype(BF16)
    l = (r - m.astype(F32)).astype(BF16)
    return h, m, l


def _gather_nn(oh_b, parts):
    h, m, l = parts
    return (_dot_nn(oh_b, h) + _dot_nn(oh_b, m)) + _dot_nn(oh_b, l)


def _gather_nt(parts, oh_b):
    h, m, l = parts
    return (_dot_nt(h, oh_b) + _dot_nt(m, oh_b)) + _dot_nt(l, oh_b)


def _gather_cols(oh_b, parts):
    h, m, l = parts
    return (_dot_nt(oh_b, h) + _dot_nt(oh_b, m)) + _dot_nt(oh_b, l)


def _erf(x):
    a1, a2, a3, a4, a5 = (0.254829592, -0.284496736, 1.421413741,
                          -1.453152027, 1.061405429)
    t = 1.0 / (1.0 + 0.3275911 * jnp.abs(x))
    poly = ((((a5 * t + a4) * t + a3) * t + a2) * t + a1) * t
    y = 1.0 - poly * jnp.exp(-x * x)
    return jnp.sign(x) * y


def _gelu(x):
    return 0.5 * x * (1.0 + _erf(x * 0.7071067811865476))


def _layer_norm(x, g, b):
    m = jnp.mean(x, axis=1, keepdims=True)
    v = jnp.mean((x - m) ** 2, axis=1, keepdims=True)
    return (x - m) / jnp.sqrt(v + 1e-5) * g + b


def _fps(coords, iota, sentinel, n_out, idx_ref, extract):
    cx, cy, cz = coords

    def step(s, carry):
        dists, far = carry
        idx_ref[pl.ds(s, 1), :] = jnp.reshape(far, (1, 1))
        sx, sy, sz = extract(far)
        d = (cx - sx) ** 2 + (cy - sy) ** 2 + (cz - sz) ** 2
        dists = jnp.minimum(dists, d)
        far = jnp.min(jnp.where(dists == jnp.max(dists), iota, sentinel))
        return dists, far

    lax.fori_loop(0, n_out, step,
                  (jnp.full(cx.shape, 1e10, F32), jnp.float32(0.0)))


def _pair_dist(cr_list, cc_list):
    n = cr_list[0].shape[1]
    d = ((cc_list[0] - cr_list[0]) ** 2
         + (cc_list[1] - cr_list[1]) ** 2
         + (cc_list[2] - cr_list[2]) ** 2)
    rio = lax.broadcasted_iota(jnp.int32, (n, n), 0)
    cio = lax.broadcasted_iota(jnp.int32, (n, n), 1)
    return d + (rio == cio).astype(F32) * 1e10


def _knn_edge(dmat, x, wmat, bias):
    n = dmat.shape[0]
    cout = wmat.shape[0]
    cio = lax.broadcasted_iota(jnp.int32, (n, n), 1).astype(F32)
    parts = _split3(x)

    def step(_, carry):
        d, m = carry
        rmin = jnp.min(d, axis=1, keepdims=True)
        nbr = jnp.min(jnp.where(d == rmin, cio, float(n)),
                      axis=1, keepdims=True)
        mask = cio == nbr
        xj = _gather_nn(mask.astype(BF16), parts)
        cat = jnp.concatenate([x, xj - x], axis=1)
        msg = _relu(_dot_nt(cat, wmat, _LO) + bias)
        return jnp.where(mask, d + 1e30, d), jnp.maximum(m, msg)

    _, m = lax.fori_loop(0, _K, step,
                         (dmat, jnp.full((n, cout), -1e30, F32)))
    return m


def _stage(coords_fps, iota_fps, coords_row, feats, n_out,
           wmat, bias, idx_ref, extract):
    n_in = coords_row[0].shape[1]
    _fps(coords_fps, iota_fps, float(n_in), n_out, idx_ref, extract)
    sel = idx_ref[0:n_out, :]
    iota_row = lax.broadcasted_iota(jnp.int32, (1, n_in), 1).astype(F32)
    onehot = (sel == iota_row).astype(F32)
    xg = _gather_nn(onehot.astype(BF16), _split3(feats))
    hi = lax.Precision.HIGHEST
    crs = [_dot_nt(c, onehot, hi) for c in coords_row]
    ccs = [_dot_nt(onehot, c, hi) for c in coords_row]
    dmat = _pair_dist(crs, ccs)
    xe = _knn_edge(dmat, xg, wmat, bias)
    return crs, jnp.concatenate([xg, xe], axis=1)


def _cross_block(s, W, i):
    u = _dot_nt(s, W(f'sa{i}_ip_w'), _LO) + W(f'sa{i}_ip_b')
    ln1 = _layer_norm(u, W(f'sa{i}_n13_g'), W(f'sa{i}_n13_b'))
    vproj = _dot_nt(ln1, W(f'sa{i}_vw'), _LO) + W(f'sa{i}_vb')
    vproj = vproj.astype(BF16).astype(F32)
    attn = _dot_nt(vproj, W(f'sa{i}_out_w'), _LO) + W(f'sa{i}_out_b')
    s1 = ln1 + attn
    ln2 = _layer_norm(s1, W(f'sa{i}_n12_g'), W(f'sa{i}_n12_b'))
    h = _gelu(_dot_nt(ln2, W(f'sa{i}_l11_w'), _LO) + W(f'sa{i}_l11_b'))
    ff = _dot_nt(h, W(f'sa{i}_l12_w'), _LO) + W(f'sa{i}_l12_b')
    return ln2 + ff


def _body(pts_ref, pts8_ref, ptsr_ref, w_refs, xg_ref, f3_ref, idx_ref):
    def W(k):
        return w_refs[k][...]

    pts = pts_ref[0]
    px = pts[0:1, :]
    py = pts[1:2, :]
    pz = pts[2:3, :]
    p8 = pts8_ref[0]
    n = px.shape[1]
    nc = n // 8

    h = _relu(_dot_nt(ptsr_ref[0], W('W1'), _LO) + W('b1'))
    x0 = _dot_nt(h, W('W2'), _LO) + W('b2')

    iota8 = (lax.broadcasted_iota(jnp.int32, (8, nc), 0) * nc
             + lax.broadcasted_iota(jnp.int32, (8, nc), 1)).astype(F32)

    def ex0(far):
        row = ptsr_ref[0, pl.ds(far.astype(jnp.int32), 1), :]
        return row[:, 0:1], row[:, 1:2], row[:, 2:3]

    def mk_ex(coords_row, iota_row):
        def ex(far):
            oh = iota_row == far
            return tuple(jnp.sum(jnp.where(oh, c, 0.0)) for c in coords_row)
        return ex

    c0, f1 = _stage((p8[0], p8[1], p8[2]), iota8, (px, py, pz), x0, n // 4,
                    W('ec1_W'), W('ec1_b'), idx_ref, ex0)
    iota1 = lax.broadcasted_iota(jnp.int32, (1, n // 4), 1).astype(F32)
    c1, f2 = _stage(tuple(c0), iota1, tuple(c0), f1, n // 8,
                    W('ec2_W'), W('ec2_b'), idx_ref, mk_ex(c0, iota1))
    iota2 = lax.broadcasted_iota(jnp.int32, (1, n // 8), 1).astype(F32)
    _, f3 = _stage(tuple(c1), iota2, tuple(c1), f2, n // 16,
                   W('ec3_W'), W('ec3_b'), idx_ref, mk_ex(c1, iota2))

    g = jnp.max(f3, axis=0, keepdims=True)
    xg_ref[0] = g

    v = _relu(_dot_nt(g, W('Wpa'), _LO) + W('bpa'))
    v = _relu(_dot_nt(v, W('Wps'), _LO) + W('bps'))
    v = _relu(_dot_nt(v, W('Wpr'), _LO) + W('bpr'))
    for i in range(3):
        v = _cross_block(v, W, i)
    c = _relu(_dot_nt(v, W('Wco1'), _LO) + W('bco1'))
    f3_ref[0] = _dot_nt(c, W('Wco'), _LO) + W('bco')


@jax.jit
def kernel(points, params):
    p = params
    B, _, N = points.shape
    E = _E

    wd = {
        'W1': p['W1'], 'b1': p['b1'].reshape(1, -1),
        'W2': p['W2'], 'b2': p['b2'].reshape(1, -1),
        'Wpa': p['Wpa'], 'bpa': p['bpa'].reshape(1, -1),
        'Wps': p['Wps'], 'bps': p['bps'].reshape(1, -1),
        'Wpr': p['Wpr'], 'bpr': p['bpr'].reshape(1, -1),
        'Wco1': p['Wco1'], 'bco1': p['bco1'].reshape(1, -1),
        'Wco': p['Wco'], 'bco': p['bco'].reshape(1, -1),
    }
    for name in ('ec1', 'ec2', 'ec3'):
        wd[f'{name}_W'] = p[f'{name}_W']
        wd[f'{name}_b'] = p[f'{name}_b'].reshape(1, -1)
    for i in range(3):
        sa = p[f'sa{i}']
        wd[f'sa{i}_ip_w'] = sa['ip_w']
        wd[f'sa{i}_ip_b'] = sa['ip_b'].reshape(1, -1)
        wd[f'sa{i}_vw'] = sa['in_w'][2 * E:3 * E]
        wd[f'sa{i}_vb'] = sa['in_b'][2 * E:3 * E].reshape(1, -1)
        wd[f'sa{i}_out_w'] = sa['out_w']
        wd[f'sa{i}_out_b'] = sa['out_b'].reshape(1, -1)
        wd[f'sa{i}_l11_w'] = sa['l11_w']
        wd[f'sa{i}_l11_b'] = sa['l11_b'].reshape(1, -1)
        wd[f'sa{i}_l12_w'] = sa['l12_w']
        wd[f'sa{i}_l12_b'] = sa['l12_b'].reshape(1, -1)
        wd[f'sa{i}_n12_g'] = sa['n12_g'].reshape(1, -1)
        wd[f'sa{i}_n12_b'] = sa['n12_b'].reshape(1, -1)
        wd[f'sa{i}_n13_g'] = sa['n13_g'].reshape(1, -1)
        wd[f'sa{i}_n13_b'] = sa['n13_b'].reshape(1, -1)

    pts8 = points.reshape(B, 3, 8, N // 8)
    ptsr = jnp.transpose(points, (0, 2, 1))

    def full_spec(a):
        return pl.BlockSpec(a.shape, lambda b, nd=a.ndim: (0,) * nd)

    in_specs = [
        pl.BlockSpec((1, 3, N), lambda b: (b, 0, 0)),
        pl.BlockSpec((1, 3, 8, N // 8), lambda b: (b, 0, 0, 0)),
        pl.BlockSpec((1, N, 3), lambda b: (b, 0, 0)),
        {k: full_spec(v) for k, v in wd.items()},
    ]
    out_specs = [
        pl.BlockSpec((1, 1, 1024), lambda b: (b, 0, 0)),
        pl.BlockSpec((1, 1, 3), lambda b: (b, 0, 0)),
    ]
    out_shape = [
        jax.ShapeDtypeStruct((B, 1, 1024), F32),
        jax.ShapeDtypeStruct((B, 1, 3), F32),
    ]
    scratch_shapes = [
        pltpu.VMEM((N // 4, 1), F32),
    ]

    xg, f3 = pl.pallas_call(
        _body,
        grid=(B,),
        in_specs=in_specs,
        out_specs=out_specs,
        out_shape=out_shape,
        scratch_shapes=scratch_shapes,
    )(points, pts8, ptsr, wd)

    fine = jnp.broadcast_to(f3[:, 0, :, None], (B, 3, 128))
    return xg[:, 0, :, None], fine


# speedup vs baseline: 3.3314x; 1.2724x over previous
import jax
import jax.numpy as jnp
from jax import lax
from jax.experimental import pallas as pl
from jax.experimental.pallas import tpu as pltpu

F32 = jnp.float32
BF16 = jnp.bfloat16
_K = 20
_E = 64
_LO = lax.Precision.DEFAULT


def _dot_nt(a, b, precision=None):
    return lax.dot_general(a, b, (((1,), (1,)), ((), ())),
                           preferred_element_type=F32, precision=precision)


def _dot_nn(a, b, precision=None):
    return lax.dot_general(a, b, (((1,), (0,)), ((), ())),
                           preferred_element_type=F32, precision=precision)


def _relu(x):
    return jnp.maximum(x, 0.0)


def _split3(v):
    h = v.astype(BF16)
    r = v - h.astype(F32)
    m = r.astype(BF16)
    l = (r - m.astype(F32)).astype(BF16)
    return h, m, l


def _gather_nn(oh_b, parts):
    h, m, l = parts
    return (_dot_nn(oh_b, h) + _dot_nn(oh_b, m)) + _dot_nn(oh_b, l)


def _gather_nt(parts, oh_b):
    h, m, l = parts
    return (_dot_nt(h, oh_b) + _dot_nt(m, oh_b)) + _dot_nt(l, oh_b)


def _gather_cols(oh_b, parts):
    h, m, l = parts
    return (_dot_nt(oh_b, h) + _dot_nt(oh_b, m)) + _dot_nt(oh_b, l)


def _erf(x):
    a1, a2, a3, a4, a5 = (0.254829592, -0.284496736, 1.421413741,
                          -1.453152027, 1.061405429)
    t = 1.0 / (1.0 + 0.3275911 * jnp.abs(x))
    poly = ((((a5 * t + a4) * t + a3) * t + a2) * t + a1) * t
    y = 1.0 - poly * jnp.exp(-x * x)
    return jnp.sign(x) * y


def _gelu(x):
    return 0.5 * x * (1.0 + _erf(x * 0.7071067811865476))


def _layer_norm(x, g, b):
    m = jnp.mean(x, axis=1, keepdims=True)
    v = jnp.mean((x - m) ** 2, axis=1, keepdims=True)
    return (x - m) / jnp.sqrt(v + 1e-5) * g + b


def _fps4(coords_list, iota, sentinel, n_out, idx_ref, extracts):
    def step(s, carry):
        new = []
        for b, ((dists, far), coords, extract) in enumerate(
                zip(carry, coords_list, extracts)):
            cx, cy, cz = coords
            idx_ref[pl.ds(b * 512 + s, 1), :] = jnp.reshape(far, (1, 1))
            sx, sy, sz = extract(far)
            d = (cx - sx) ** 2 + (cy - sy) ** 2 + (cz - sz) ** 2
            dists = jnp.minimum(dists, d)
            far = jnp.min(jnp.where(dists == jnp.max(dists), iota, sentinel))
            new.append((dists, far))
        return tuple(new)

    init = tuple((jnp.full(c[0].shape, 1e10, F32), jnp.float32(0.0))
                 for c in coords_list)
    lax.fori_loop(0, n_out, step, init)


def _pair_dist(cr_list, cc_list):
    n = cr_list[0].shape[1]
    d = ((cc_list[0] - cr_list[0]) ** 2
         + (cc_list[1] - cr_list[1]) ** 2
         + (cc_list[2] - cr_list[2]) ** 2)
    rio = lax.broadcasted_iota(jnp.int32, (n, n), 0)
    cio = lax.broadcasted_iota(jnp.int32, (n, n), 1)
    return d + (rio == cio).astype(F32) * 1e10


def _knn_edge4(dmats, xs, wmat, bias):
    n = dmats[0].shape[0]
    cout = wmat.shape[0]
    cio = lax.broadcasted_iota(jnp.int32, (n, n), 1).astype(F32)
    parts = [_split3(x) for x in xs]

    def step(_, carry):
        new = []
        for b, (d, m) in enumerate(carry):
            rmin = jnp.min(d, axis=1, keepdims=True)
            nbr = jnp.min(jnp.where(d == rmin, cio, float(n)),
                          axis=1, keepdims=True)
            mask = cio == nbr
            xj = _gather_nn(mask.astype(BF16), parts[b])
            cat = jnp.concatenate([xs[b], xj - xs[b]], axis=1)
            msg = _relu(_dot_nt(cat, wmat, _LO) + bias)
            new.append((jnp.where(mask, d + 1e30, d), jnp.maximum(m, msg)))
        return tuple(new)

    init = tuple((d, jnp.full((n, cout), -1e30, F32)) for d in dmats)
    out = lax.fori_loop(0, _K, step, init)
    return [m for _, m in out]


def _stage4(coords_fps_list, iota_fps, coords_row_list, feats_list, n_out,
            wmat, bias, idx_ref, extracts):
    n_in = coords_row_list[0][0].shape[1]
    _fps4(coords_fps_list, iota_fps, float(n_in), n_out, idx_ref, extracts)
    iota_row = lax.broadcasted_iota(jnp.int32, (1, n_in), 1).astype(F32)
    hi = lax.Precision.HIGHEST
    crs_list, xgs, dmats = [], [], []
    for b, (coords_row, feats) in enumerate(zip(coords_row_list, feats_list)):
        sel = idx_ref[b * 512:b * 512 + n_out, :]
        onehot = (sel == iota_row).astype(F32)
        xgs.append(_gather_nn(onehot.astype(BF16), _split3(feats)))
        crs = [_dot_nt(c, onehot, hi) for c in coords_row]
        ccs = [_dot_nt(onehot, c, hi) for c in coords_row]
        crs_list.append(crs)
        dmats.append(_pair_dist(crs, ccs))
    xes = _knn_edge4(dmats, xgs, wmat, bias)
    outs = [jnp.concatenate([xg, xe], axis=1) for xg, xe in zip(xgs, xes)]
    return crs_list, outs


def _cross_block(s, W, i):
    u = _dot_nt(s, W(f'sa{i}_ip_w'), _LO) + W(f'sa{i}_ip_b')
    ln1 = _layer_norm(u, W(f'sa{i}_n13_g'), W(f'sa{i}_n13_b'))
    vproj = _dot_nt(ln1, W(f'sa{i}_vw'), _LO) + W(f'sa{i}_vb')
    vproj = vproj.astype(BF16).astype(F32)
    attn = _dot_nt(vproj, W(f'sa{i}_out_w'), _LO) + W(f'sa{i}_out_b')
    s1 = ln1 + attn
    ln2 = _layer_norm(s1, W(f'sa{i}_n12_g'), W(f'sa{i}_n12_b'))
    h = _gelu(_dot_nt(ln2, W(f'sa{i}_l11_w'), _LO) + W(f'sa{i}_l11_b'))
    ff = _dot_nt(h, W(f'sa{i}_l12_w'), _LO) + W(f'sa{i}_l12_b')
    return ln2 + ff


def _body(pts_ref, pts8_ref, ptsr_ref, w_refs, xg_ref, f3_ref, idx_ref):
    def W(k):
        return w_refs[k][...]

    B = pts_ref.shape[0]
    n = pts_ref.shape[2]
    nc = n // 8

    h = _relu(_dot_nt(ptsr_ref[...], W('W1'), _LO) + W('b1'))
    x0 = _dot_nt(h, W('W2'), _LO) + W('b2')

    iota8 = (lax.broadcasted_iota(jnp.int32, (8, nc), 0) * nc
             + lax.broadcasted_iota(jnp.int32, (8, nc), 1)).astype(F32)

    def mk_ex0(b):
        def ex(far):
            row = ptsr_ref[pl.ds(b * n + far.astype(jnp.int32), 1), :]
            return row[:, 0:1], row[:, 1:2], row[:, 2:3]
        return ex

    def mk_ex(coords_row, iota_row):
        def ex(far):
            oh = iota_row == far
            return tuple(jnp.sum(jnp.where(oh, c, 0.0)) for c in coords_row)
        return ex

    cf0, cr0, f0, ex0 = [], [], [], []
    for b in range(B):
        p8 = pts8_ref[b]
        pts = pts_ref[b]
        cf0.append((p8[0], p8[1], p8[2]))
        cr0.append((pts[0:1, :], pts[1:2, :], pts[2:3, :]))
        f0.append(x0[b * n:(b + 1) * n, :])
        ex0.append(mk_ex0(b))

    c0, f1 = _stage4(cf0, iota8, cr0, f0, n // 4,
                     W('ec1_W'), W('ec1_b'), idx_ref, ex0)
    iota1 = lax.broadcasted_iota(jnp.int32, (1, n // 4), 1).astype(F32)
    c1, f2 = _stage4([tuple(c) for c in c0], iota1, [tuple(c) for c in c0],
                     f1, n // 8, W('ec2_W'), W('ec2_b'), idx_ref,
                     [mk_ex(c, iota1) for c in c0])
    iota2 = lax.broadcasted_iota(jnp.int32, (1, n // 8), 1).astype(F32)
    _, f3 = _stage4([tuple(c) for c in c1], iota2, [tuple(c) for c in c1],
                    f2, n // 16, W('ec3_W'), W('ec3_b'), idx_ref,
                    [mk_ex(c, iota2) for c in c1])

    g = jnp.concatenate([jnp.max(fb, axis=0, keepdims=True) for fb in f3],
                        axis=0)
    xg_ref[...] = g

    v = _relu(_dot_nt(g, W('Wpa'), _LO) + W('bpa'))
    v = _relu(_dot_nt(v, W('Wps'), _LO) + W('bps'))
    v = _relu(_dot_nt(v, W('Wpr'), _LO) + W('bpr'))
    for i in range(3):
        v = _cross_block(v, W, i)
    c = _relu(_dot_nt(v, W('Wco1'), _LO) + W('bco1'))
    f3_ref[...] = _dot_nt(c, W('Wco'), _LO) + W('bco')


@jax.jit
def kernel(points, params):
    p = params
    B, _, N = points.shape
    E = _E

    wd = {
        'W1': p['W1'], 'b1': p['b1'].reshape(1, -1),
        'W2': p['W2'], 'b2': p['b2'].reshape(1, -1),
        'Wpa': p['Wpa'], 'bpa': p['bpa'].reshape(1, -1),
        'Wps': p['Wps'], 'bps': p['bps'].reshape(1, -1),
        'Wpr': p['Wpr'], 'bpr': p['bpr'].reshape(1, -1),
        'Wco1': p['Wco1'], 'bco1': p['bco1'].reshape(1, -1),
        'Wco': p['Wco'], 'bco': p['bco'].reshape(1, -1),
    }
    for name in ('ec1', 'ec2', 'ec3'):
        wd[f'{name}_W'] = p[f'{name}_W']
        wd[f'{name}_b'] = p[f'{name}_b'].reshape(1, -1)
    for i in range(3):
        sa = p[f'sa{i}']
        wd[f'sa{i}_ip_w'] = sa['ip_w']
        wd[f'sa{i}_ip_b'] = sa['ip_b'].reshape(1, -1)
        wd[f'sa{i}_vw'] = sa['in_w'][2 * E:3 * E]
        wd[f'sa{i}_vb'] = sa['in_b'][2 * E:3 * E].reshape(1, -1)
        wd[f'sa{i}_out_w'] = sa['out_w']
        wd[f'sa{i}_out_b'] = sa['out_b'].reshape(1, -1)
        wd[f'sa{i}_l11_w'] = sa['l11_w']
        wd[f'sa{i}_l11_b'] = sa['l11_b'].reshape(1, -1)
        wd[f'sa{i}_l12_w'] = sa['l12_w']
        wd[f'sa{i}_l12_b'] = sa['l12_b'].reshape(1, -1)
        wd[f'sa{i}_n12_g'] = sa['n12_g'].reshape(1, -1)
        wd[f'sa{i}_n12_b'] = sa['n12_b'].reshape(1, -1)
        wd[f'sa{i}_n13_g'] = sa['n13_g'].reshape(1, -1)
        wd[f'sa{i}_n13_b'] = sa['n13_b'].reshape(1, -1)

    pts8 = points.reshape(B, 3, 8, N // 8)
    ptsr = jnp.transpose(points, (0, 2, 1)).reshape(B * N, 3)

    out_shape = [
        jax.ShapeDtypeStruct((B, 1024), F32),
        jax.ShapeDtypeStruct((B, 3), F32),
    ]
    scratch_shapes = [
        pltpu.VMEM((B * 512, 1), F32),
    ]

    xg, f3 = pl.pallas_call(
        _body,
        out_shape=out_shape,
        scratch_shapes=scratch_shapes,
    )(points, pts8, ptsr, wd)

    fine = jnp.broadcast_to(f3[:, :, None], (B, 3, 128))
    return xg[:, :, None], fine
